# baseline (device time: 18545 ns/iter reference)
import jax
import jax.numpy as jnp
from jax import lax
from jax.experimental import pallas as pl
from jax.experimental.pallas import tpu as pltpu

N_DEV = 16


def kernel(x, w_mat):
    m_total, k_shard = x.shape
    n = w_mat.shape[1]
    m_blk = m_total // N_DEV

    x = pltpu.with_memory_space_constraint(x, pltpu.MemorySpace.HBM)
    w_mat = pltpu.with_memory_space_constraint(
        w_mat, pltpu.MemorySpace.HBM
    )

    def body(x_hbm, w_hbm, o_hbm, x_vmem, xrow_ref, w_vmem, acc_ref,
             send_sems, recv_sems, w_sems, ready_sems, x_sem, o_sem):
        my = lax.axis_index("i")

        barrier_sem = pltpu.get_barrier_semaphore()
        pl.semaphore_signal(barrier_sem, inc=1)
        pl.semaphore_wait(barrier_sem, 1)

        x_cp = pltpu.make_async_copy(x_hbm, x_vmem, x_sem)
        x_cp.start()

        for d in range(1, N_DEV):
            tgt = lax.rem(my + d, N_DEV)
            pl.semaphore_signal(
                ready_sems.at[N_DEV - d], inc=1,
                device_id=(tgt,), device_id_type=pl.DeviceIdType.MESH,
            )

        x_cp.wait()
        sends = []
        for d in range(1, N_DEV):
            tgt = lax.rem(my + d, N_DEV)
            pl.semaphore_wait(ready_sems.at[d], 1)
            rdma = pltpu.make_async_remote_copy(
                src_ref=x_vmem.at[pl.ds(tgt * m_blk, m_blk), :],
                dst_ref=xrow_ref.at[:, pl.ds(my * k_shard, k_shard)],
                send_sem=send_sems.at[d],
                recv_sem=recv_sems.at[d],
                device_id=(tgt,),
                device_id_type=pl.DeviceIdType.MESH,
            )
            rdma.start()
            sends.append(rdma)

        w_copies = []
        for d in range(N_DEV):
            src_dev = lax.rem(my - d + N_DEV, N_DEV)
            cp = pltpu.make_async_copy(
                w_hbm.at[pl.ds(src_dev * k_shard, k_shard), :],
                w_vmem.at[d],
                w_sems.at[d],
            )
            cp.start()
            w_copies.append(cp)

        w_copies[0].wait()
        own = x_vmem[pl.ds(my * m_blk, m_blk), :]
        acc_ref[...] = jnp.dot(
            own, w_vmem[0], preferred_element_type=jnp.float32,
        )

        for d in range(1, N_DEV):
            src_dev = lax.rem(my - d + N_DEV, N_DEV)
            recv = pltpu.make_async_remote_copy(
                src_ref=x_vmem.at[pl.ds(0, m_blk), :],
                dst_ref=xrow_ref.at[:, pl.ds(src_dev * k_shard, k_shard)],
                send_sem=send_sems.at[0],
                recv_sem=recv_sems.at[d],
                device_id=(my,),
                device_id_type=pl.DeviceIdType.MESH,
            )
            recv.wait_recv()
            w_copies[d].wait()
            chunk = xrow_ref[:, pl.ds(src_dev * k_shard, k_shard)]
            acc_ref[...] = acc_ref[...] + jnp.dot(
                chunk, w_vmem[d], preferred_element_type=jnp.float32,
            )

        acc_ref[...] = jnp.maximum(acc_ref[...], 0.0)
        o_cp = pltpu.make_async_copy(acc_ref, o_hbm, o_sem)
        o_cp.start()
        o_cp.wait()

        for rdma in sends:
            rdma.wait_send()

    return pl.pallas_call(
        body,
        out_shape=jax.ShapeDtypeStruct((m_blk, n), jnp.float32),
        in_specs=[
            pl.BlockSpec(memory_space=pl.ANY),
            pl.BlockSpec(memory_space=pl.ANY),
        ],
        out_specs=pl.BlockSpec(memory_space=pltpu.MemorySpace.HBM),
        scratch_shapes=[
            pltpu.VMEM((m_total, k_shard), jnp.float32),
            pltpu.VMEM((m_blk, m_total), jnp.float32),
            pltpu.VMEM((N_DEV, k_shard, n), jnp.float32),
            pltpu.VMEM((m_blk, n), jnp.float32),
            pltpu.SemaphoreType.DMA((N_DEV,)),
            pltpu.SemaphoreType.DMA((N_DEV,)),
            pltpu.SemaphoreType.DMA((N_DEV,)),
            pltpu.SemaphoreType.REGULAR((N_DEV,)),
            pltpu.SemaphoreType.DMA,
            pltpu.SemaphoreType.DMA,
        ],
        compiler_params=pltpu.CompilerParams(collective_id=0),
    )(x, w_mat)


# device time: 18395 ns/iter; 1.0082x vs baseline; 1.0082x over previous
import jax
import jax.numpy as jnp
from jax import lax
from jax.experimental import pallas as pl
from jax.experimental.pallas import tpu as pltpu

N_DEV = 16


def kernel(x, w_mat):
    m_total, k_shard = x.shape
    n = w_mat.shape[1]
    m_blk = m_total // N_DEV

    x = pltpu.with_memory_space_constraint(x, pltpu.MemorySpace.HBM)
    w_mat = pltpu.with_memory_space_constraint(
        w_mat, pltpu.MemorySpace.HBM
    )

    def body(x_hbm, w_hbm, o_hbm, x_vmem, xrow_ref, w_vmem, acc_ref,
             send_sems, recv_sems, w_sems, x_sem, o_sem):
        my = lax.axis_index("i")

        x_cp = pltpu.make_async_copy(x_hbm, x_vmem, x_sem)
        x_cp.start()

        barrier_sem = pltpu.get_barrier_semaphore()
        for d in range(1, N_DEV):
            nbr = lax.rem(my + d, N_DEV)
            pl.semaphore_signal(
                barrier_sem, inc=1,
                device_id=(nbr,), device_id_type=pl.DeviceIdType.MESH,
            )
        pl.semaphore_wait(barrier_sem, N_DEV - 1)

        x_cp.wait()
        sends = []
        for d in range(1, N_DEV):
            tgt = lax.rem(my + d, N_DEV)
            rdma = pltpu.make_async_remote_copy(
                src_ref=x_vmem.at[pl.ds(tgt * m_blk, m_blk), :],
                dst_ref=xrow_ref.at[:, pl.ds(my * k_shard, k_shard)],
                send_sem=send_sems.at[d],
                recv_sem=recv_sems.at[d],
                device_id=(tgt,),
                device_id_type=pl.DeviceIdType.MESH,
            )
            rdma.start()
            sends.append(rdma)

        w_copies = []
        for d in range(N_DEV):
            src_dev = lax.rem(my - d + N_DEV, N_DEV)
            cp = pltpu.make_async_copy(
                w_hbm.at[pl.ds(src_dev * k_shard, k_shard), :],
                w_vmem.at[d],
                w_sems.at[d],
            )
            cp.start()
            w_copies.append(cp)

        w_copies[0].wait()
        own = x_vmem[pl.ds(my * m_blk, m_blk), :]
        acc_ref[...] = jnp.dot(
            own, w_vmem[0], preferred_element_type=jnp.float32,
        )

        for d in range(1, N_DEV):
            src_dev = lax.rem(my - d + N_DEV, N_DEV)
            recv = pltpu.make_async_remote_copy(
                src_ref=x_vmem.at[pl.ds(0, m_blk), :],
                dst_ref=xrow_ref.at[:, pl.ds(src_dev * k_shard, k_shard)],
                send_sem=send_sems.at[0],
                recv_sem=recv_sems.at[d],
                device_id=(my,),
                device_id_type=pl.DeviceIdType.MESH,
            )
            recv.wait_recv()
            w_copies[d].wait()
            chunk = xrow_ref[:, pl.ds(src_dev * k_shard, k_shard)]
            acc_ref[...] = acc_ref[...] + jnp.dot(
                chunk, w_vmem[d], preferred_element_type=jnp.float32,
            )

        acc_ref[...] = jnp.maximum(acc_ref[...], 0.0)
        o_cp = pltpu.make_async_copy(acc_ref, o_hbm, o_sem)
        o_cp.start()

        for rdma in sends:
            rdma.wait_send()
        o_cp.wait()

    return pl.pallas_call(
        body,
        out_shape=jax.ShapeDtypeStruct((m_blk, n), jnp.float32),
        in_specs=[
            pl.BlockSpec(memory_space=pl.ANY),
            pl.BlockSpec(memory_space=pl.ANY),
        ],
        out_specs=pl.BlockSpec(memory_space=pl.ANY),
        scratch_shapes=[
            pltpu.VMEM((m_total, k_shard), jnp.float32),
            pltpu.VMEM((m_blk, m_total), jnp.float32),
            pltpu.VMEM((N_DEV, k_shard, n), jnp.float32),
            pltpu.VMEM((m_blk, n), jnp.float32),
            pltpu.SemaphoreType.DMA((N_DEV,)),
            pltpu.SemaphoreType.DMA((N_DEV,)),
            pltpu.SemaphoreType.DMA((N_DEV,)),
            pltpu.SemaphoreType.DMA,
            pltpu.SemaphoreType.DMA,
        ],
        compiler_params=pltpu.CompilerParams(collective_id=0),
    )(x, w_mat)
